# Initial kernel scaffold; baseline (speedup 1.0000x reference)
import jax
import jax.numpy as jnp
from jax.experimental import pallas as pl


def kernel(x, gate_W, gate_b, expert_W, expert_b):
    raise NotImplementedError("write your pallas kernel here")


# speedup vs baseline: 2.9579x; 2.9579x over previous
import functools

import jax
import jax.numpy as jnp
from jax.experimental import pallas as pl
from jax.experimental.pallas import tpu as pltpu

_NEG_INF = -1e30


def _moe_body(x_ref, gw_ref, gb_ref, ew_ref, eb_ref, out_ref, *, n_experts):
    xt = x_ref[...]
    logits = jax.lax.dot_general(
        xt, gw_ref[...], (((1,), (1,)), ((), ())),
        preferred_element_type=jnp.float32,
    ) + gb_ref[...]
    w = jax.nn.softmax(logits, axis=-1)
    eidx = jax.lax.broadcasted_iota(jnp.int32, w.shape, 1)
    i1 = jnp.argmax(w, axis=1)[:, None]
    v1 = jnp.max(w, axis=1, keepdims=True)
    wm = jnp.where(eidx == i1, _NEG_INF, w)
    i2 = jnp.argmax(wm, axis=1)[:, None]
    v2 = jnp.max(wm, axis=1, keepdims=True)
    r = jnp.exp(v2 - v1)
    c1 = 1.0 / (1.0 + r)
    c2 = r / (1.0 + r)
    acc = jnp.zeros_like(xt)
    for e in range(n_experts):
        y = jax.lax.dot_general(
            xt, ew_ref[e], (((1,), (1,)), ((), ())),
            preferred_element_type=jnp.float32,
        ) + eb_ref[e][None, :]
        c_e = jnp.where(i1 == e, c1, jnp.where(i2 == e, c2, 0.0))
        acc = acc + c_e * y
    out_ref[...] = acc


def kernel(x, gate_W, gate_b, expert_W, expert_b):
    N, d = x.shape
    E = gate_W.shape[0]
    TN = 512
    grid = (N // TN,)
    gate_b2 = gate_b.reshape(1, E)
    return pl.pallas_call(
        functools.partial(_moe_body, n_experts=E),
        grid=grid,
        in_specs=[
            pl.BlockSpec((TN, d), lambda i: (i, 0)),
            pl.BlockSpec((E, d), lambda i: (0, 0)),
            pl.BlockSpec((1, E), lambda i: (0, 0)),
            pl.BlockSpec((E, d, d), lambda i: (0, 0, 0)),
            pl.BlockSpec((E, d), lambda i: (0, 0)),
        ],
        out_specs=pl.BlockSpec((TN, d), lambda i: (i, 0)),
        out_shape=jax.ShapeDtypeStruct((N, d), x.dtype),
        compiler_params=pltpu.CompilerParams(
            dimension_semantics=("arbitrary",),
        ),
    )(x, gate_W, gate_b2, expert_W, expert_b)
